# baseline (speedup 1.0000x reference)
import functools

import jax
import jax.numpy as jnp
from jax import lax
from jax.experimental import pallas as pl
from jax.experimental.pallas import tpu as pltpu
from jax.experimental.pallas import tpu_sc as plsc

B, L, V, D, O = 16384, 50, 1000000, 64, 5
NC, NS, LANES = 2, 16, 16
NW = NC * NS
SEG = 3 * B
SEGW = SEG // NW
G = 96
NBUF = 12
KG = G // NBUF
SG = 384
GL = 56
NSG = SEGW // SG
INV_L = 1.0 / L


def _pool_body(idx_hbm, table_hbm, out_hbm, idx_v, rows_v, pooled_v, osem0, osem1, *sems):
    w = lax.axis_index("s") * NC + lax.axis_index("c")
    seg0 = w * SEGW

    def _issue(slot, s):
        pltpu.async_copy(
            table_hbm.at[idx_v.at[s, pl.ds(0, GL)]], rows_v.at[slot], sems[slot]
        )

    def _wait(slot):
        pltpu.make_async_copy(
            table_hbm.at[idx_v.at[0, pl.ds(0, GL)]], rows_v.at[slot], sems[slot]
        ).wait()

    def _owait(osem):
        pltpu.make_async_copy(
            pooled_v.at[pl.ds(0, G)], out_hbm.at[pl.ds(seg0, G)], osem
        ).wait()

    def _bf(v):
        return plsc.bitcast(v, jnp.bfloat16)

    lo, hi = pl.ds(0, LANES), pl.ds(LANES, LANES)

    def _super(sg, carry):
        sbase = seg0 + sg * SG
        pltpu.sync_copy(idx_hbm.at[pl.ds(sbase, SG)], idx_v)
        for b in range(NBUF):
            _issue(b, b)

        def _ring(i, carry):
            s = i * NBUF
            so = lax.rem(s, 2 * G)
            for b in range(NBUF):
                _wait(b)
                a0 = _bf(rows_v[b, 0, lo])
                b0 = _bf(rows_v[b, 0, hi])
                a1 = _bf(rows_v[b, 1, lo])
                b1 = _bf(rows_v[b, 1, hi])
                for r in range(2, L, 2):
                    a0 = a0 + _bf(rows_v[b, r, lo])
                    b0 = b0 + _bf(rows_v[b, r, hi])
                    a1 = a1 + _bf(rows_v[b, r + 1, lo])
                    b1 = b1 + _bf(rows_v[b, r + 1, hi])
                s0, s2 = plsc.unpack(a0 + a1, format=plsc.PackFormat.INTERLEAVED)
                s1, s3 = plsc.unpack(b0 + b1, format=plsc.PackFormat.INTERLEAVED)
                pooled_v[so + b, pl.ds(0 * LANES, LANES)] = s0 * INV_L
                pooled_v[so + b, pl.ds(1 * LANES, LANES)] = s1 * INV_L
                pooled_v[so + b, pl.ds(2 * LANES, LANES)] = s2 * INV_L
                pooled_v[so + b, pl.ds(3 * LANES, LANES)] = s3 * INV_L
                nxt = s + b + NBUF

                @pl.when(nxt < SG)
                def _():
                    _issue(b, nxt)

            @pl.when(lax.rem(i + 1, KG) == 0)
            def _():
                lg = (i + 1) // KG - 1
                gi = sg * (SG // G) + lg
                par = lax.rem(lg, 2)
                dst = out_hbm.at[pl.ds(sbase + lg * G, G)]

                @pl.when((gi >= 1) & (par == 0))
                def _():
                    _owait(osem1)

                @pl.when((gi >= 1) & (par == 1))
                def _():
                    _owait(osem0)

                @pl.when(par == 0)
                def _():
                    pltpu.async_copy(pooled_v.at[pl.ds(0, G)], dst, osem0)

                @pl.when(par == 1)
                def _():
                    pltpu.async_copy(pooled_v.at[pl.ds(G, G)], dst, osem1)

            return carry

        lax.fori_loop(0, SG // NBUF, _ring, 0)
        return carry

    lax.fori_loop(0, NSG, _super, 0)
    _owait(osem1)


def _pool(idx_all, emb_table):
    mesh = plsc.VectorSubcoreMesh(core_axis_name="c", subcore_axis_name="s")
    return pl.kernel(
        _pool_body,
        out_type=jax.ShapeDtypeStruct((SEG, D), jnp.float32),
        mesh=mesh,
        scratch_types=[
            pltpu.VMEM((SG, 128), jnp.int32),
            pltpu.VMEM((NBUF, GL, D // 2), jnp.uint32),
            pltpu.VMEM((2 * G, D), jnp.float32),
            pltpu.SemaphoreType.DMA,
            pltpu.SemaphoreType.DMA,
        ]
        + [pltpu.SemaphoreType.DMA] * NBUF,
        compiler_params=pltpu.CompilerParams(
            use_tc_tiling_on_sc=False, needs_layout_passes=False
        ),
    )(idx_all, emb_table)


def _packtab_body(in_ref, out_ref):
    x = in_ref[...]
    u = lax.bitcast_convert_type(x, jnp.uint32) + jnp.uint32(0x8000)
    lo = u[:, : D // 2] >> jnp.uint32(16)
    hi = u[:, D // 2 :] & jnp.uint32(0xFFFF0000)
    val = lo | hi
    y = val.reshape(val.shape[0] // 4, 4, D // 2)
    for q in range(4):
        out_ref[:, q * 32 : (q + 1) * 32] = y[:, q]


def _packtab(emb_table, rt=8000):
    return pl.pallas_call(
        _packtab_body,
        grid=(V // rt,),
        in_specs=[pl.BlockSpec((rt, D), lambda i: (i, 0))],
        out_specs=pl.BlockSpec((rt // 4, 128), lambda i: (i, 0)),
        out_shape=jax.ShapeDtypeStruct((V * (D // 2) // 128, 128), jnp.uint32),
    )(emb_table)


def _head_body(pooled_ref, mw_ref, clfw_ref, clfb_ref, out_ref):
    mw = mw_ref[...]
    fw = clfw_ref[...]
    logits = clfb_ref[...]
    for wdx in range(3):
        f = jnp.dot(
            fw[:, wdx * D : (wdx + 1) * D], mw, preferred_element_type=jnp.float32
        )
        logits = logits + jnp.dot(
            pooled_ref[wdx], f.T, preferred_element_type=jnp.float32
        )
    m = jnp.max(logits, axis=1, keepdims=True)
    e = jnp.exp(logits - m)
    out_ref[...] = e / jnp.sum(e, axis=1, keepdims=True)


def _head(pooled, m_w, clf_w, clf_b, bm=4096):
    return pl.pallas_call(
        _head_body,
        grid=(B // bm,),
        in_specs=[
            pl.BlockSpec((3, bm, D), lambda i: (0, i, 0)),
            pl.BlockSpec((D, D), lambda i: (0, 0)),
            pl.BlockSpec((O, 3 * D), lambda i: (0, 0)),
            pl.BlockSpec((1, O), lambda i: (0, 0)),
        ],
        out_specs=pl.BlockSpec((bm, O), lambda i: (i, 0)),
        out_shape=jax.ShapeDtypeStruct((B, O), jnp.float32),
    )(pooled, m_w, clf_w, clf_b)


def kernel(left_idx, term_idx, right_idx, emb_table, m_w, clf_w, clf_b):
    idx_all = jnp.concatenate(
        [
            left_idx.astype(jnp.int32),
            term_idx.astype(jnp.int32),
            right_idx.astype(jnp.int32),
        ],
        axis=0,
    )
    idx_pad = jnp.pad(idx_all, ((0, 0), (0, 128 - L)))
    packed = _packtab(emb_table).reshape(V, D // 2)
    pooled = _pool(idx_pad, packed).reshape(3, B, D)
    return _head(pooled, m_w, clf_w, clf_b.reshape(1, O))


# speedup vs baseline: 4.5416x; 4.5416x over previous
import functools

import jax
import jax.numpy as jnp
from jax import lax
from jax.experimental import pallas as pl
from jax.experimental.pallas import tpu as pltpu
from jax.experimental.pallas import tpu_sc as plsc

B, L, V, D, O = 16384, 50, 1000000, 64, 5
NC, NS, LANES = 2, 16, 16
NW = NC * NS
SEG = 3 * B
SEGW = SEG // NW
G = 96
NBUF = 8
KG = G // NBUF
INV_L = 1.0 / L


def _pool_body(idx_hbm, table_hbm, out_hbm, idx_v, rows_v, pooled_v, *sems):
    w = lax.axis_index("s") * NC + lax.axis_index("c")
    seg0 = w * SEGW

    def _issue(slot, s):
        pltpu.async_copy(table_hbm.at[idx_v.at[s]], rows_v.at[slot], sems[slot])

    def _wait(slot):
        pltpu.make_async_copy(
            table_hbm.at[idx_v.at[0]], rows_v.at[slot], sems[slot]
        ).wait()

    pltpu.sync_copy(idx_hbm.at[pl.ds(seg0, SEGW)], idx_v)
    for b in range(NBUF):
        _issue(b, b)

    def _bf(v):
        return plsc.bitcast(v, jnp.bfloat16)

    lo, hi = pl.ds(0, LANES), pl.ds(LANES, LANES)

    def _ring(i, carry):
        s = i * NBUF
        so = lax.rem(s, G)
        for b in range(NBUF):
            _wait(b)
            a0 = _bf(rows_v[b, 0, lo])
            b0 = _bf(rows_v[b, 0, hi])
            a1 = _bf(rows_v[b, 1, lo])
            b1 = _bf(rows_v[b, 1, hi])
            for r in range(2, L, 2):
                a0 = a0 + _bf(rows_v[b, r, lo])
                b0 = b0 + _bf(rows_v[b, r, hi])
                a1 = a1 + _bf(rows_v[b, r + 1, lo])
                b1 = b1 + _bf(rows_v[b, r + 1, hi])
            s0, s2 = plsc.unpack(a0 + a1, format=plsc.PackFormat.INTERLEAVED)
            s1, s3 = plsc.unpack(b0 + b1, format=plsc.PackFormat.INTERLEAVED)
            pooled_v[so + b, pl.ds(0 * LANES, LANES)] = s0 * INV_L
            pooled_v[so + b, pl.ds(1 * LANES, LANES)] = s1 * INV_L
            pooled_v[so + b, pl.ds(2 * LANES, LANES)] = s2 * INV_L
            pooled_v[so + b, pl.ds(3 * LANES, LANES)] = s3 * INV_L
            nxt = s + b + NBUF

            @pl.when(nxt < SEGW)
            def _():
                _issue(b, nxt)

        @pl.when(lax.rem(i + 1, KG) == 0)
        def _():
            g0 = (i + 1 - KG) * NBUF
            pltpu.sync_copy(pooled_v, out_hbm.at[pl.ds(seg0 + g0, G)])

        return carry

    lax.fori_loop(0, SEGW // NBUF, _ring, 0)


def _pool(idx_all, emb_table):
    mesh = plsc.VectorSubcoreMesh(core_axis_name="c", subcore_axis_name="s")
    return pl.kernel(
        _pool_body,
        out_type=jax.ShapeDtypeStruct((SEG, D), jnp.float32),
        mesh=mesh,
        scratch_types=[
            pltpu.VMEM((SEGW, L), jnp.int32),
            pltpu.VMEM((NBUF, L, D // 2), jnp.uint32),
            pltpu.VMEM((G, D), jnp.float32),
        ]
        + [pltpu.SemaphoreType.DMA] * NBUF,
        compiler_params=pltpu.CompilerParams(
            use_tc_tiling_on_sc=False, needs_layout_passes=False
        ),
    )(idx_all, emb_table)


TROW = 31232
PCR = 128
PNB = 4


def _scpack_body(tab_hbm, out_hbm, in_v, out_v, *sems):
    isems, osems = sems[:PNB], sems[PNB:]
    w = lax.axis_index("s") * NC + lax.axis_index("c")
    base = w * TROW
    last = w == NW - 1

    def _issue_in(slot, c):
        off = pl.multiple_of(base + c * PCR, 8)
        pltpu.async_copy(tab_hbm.at[pl.ds(off, PCR)], in_v.at[slot], isems[slot])

    def _wait_in(slot):
        pltpu.make_async_copy(
            tab_hbm.at[pl.ds(0, PCR)], in_v.at[slot], isems[slot]
        ).wait()

    def _wait_out(slot):
        pltpu.make_async_copy(
            out_v.at[slot], out_hbm.at[pl.ds(0, PCR // 4)], osems[slot]
        ).wait()

    def _pack_rows(slot, nrows):
        for r in range(nrows):
            a0 = in_v[slot, r, pl.ds(0, LANES)]
            a1 = in_v[slot, r, pl.ds(LANES, LANES)]
            b0 = in_v[slot, r, pl.ds(2 * LANES, LANES)]
            b1 = in_v[slot, r, pl.ds(3 * LANES, LANES)]
            w0 = plsc.bitcast(
                plsc.pack(a0, b0, format=plsc.PackFormat.INTERLEAVED), jnp.uint32
            )
            w1 = plsc.bitcast(
                plsc.pack(a1, b1, format=plsc.PackFormat.INTERLEAVED), jnp.uint32
            )
            out_v[slot, r // 4, pl.ds(32 * (r % 4), LANES)] = w0
            out_v[slot, r // 4, pl.ds(32 * (r % 4) + LANES, LANES)] = w1

    nch = TROW // PCR + ((V - TROW * NW) // PCR) * jnp.where(last, 1, 0)
    for b in range(PNB):
        _issue_in(b, b)

    def _chunk(i, carry):
        for b in range(PNB):
            c = i * PNB + b

            @pl.when(c >= PNB)
            def _():
                _wait_out(b)

            _wait_in(b)
            _pack_rows(b, PCR)
            ooff = pl.multiple_of((base + c * PCR) // 4, 8)
            pltpu.async_copy(
                out_v.at[b], out_hbm.at[pl.ds(ooff, PCR // 4)], osems[b]
            )

            @pl.when(c + PNB < nch)
            def _():
                _issue_in(b, c + PNB)

        return carry

    lax.fori_loop(0, nch // PNB, _chunk, 0)
    for b in range(PNB):
        _wait_out(b)

    @pl.when(last)
    def _():
        t0 = NW * TROW + ((V - TROW * NW) // PCR) * PCR
        pltpu.sync_copy(tab_hbm.at[pl.ds(t0, 64)], in_v.at[0, pl.ds(0, 64)])
        _pack_rows(0, 64)
        pltpu.sync_copy(out_v.at[0, pl.ds(0, 16)], out_hbm.at[pl.ds(t0 // 4, 16)])


def _scpack(emb_table):
    mesh = plsc.VectorSubcoreMesh(core_axis_name="c", subcore_axis_name="s")
    return pl.kernel(
        _scpack_body,
        out_type=jax.ShapeDtypeStruct((V * (D // 2) // 128, 128), jnp.uint32),
        mesh=mesh,
        scratch_types=[
            pltpu.VMEM((PNB, PCR, D), jnp.float32),
            pltpu.VMEM((PNB, PCR // 4, 128), jnp.uint32),
        ]
        + [pltpu.SemaphoreType.DMA] * (2 * PNB),
        compiler_params=pltpu.CompilerParams(
            use_tc_tiling_on_sc=True, needs_layout_passes=False
        ),
    )(emb_table)


def _packtab_body(in_ref, out_ref):
    x = in_ref[...]
    u = lax.bitcast_convert_type(x, jnp.uint32) + jnp.uint32(0x8000)
    lo = u[:, : D // 2] >> jnp.uint32(16)
    hi = u[:, D // 2 :] & jnp.uint32(0xFFFF0000)
    val = lo | hi
    y = val.reshape(val.shape[0] // 4, 4, D // 2)
    for q in range(4):
        out_ref[:, q * 32 : (q + 1) * 32] = y[:, q]


def _packtab(emb_table, rt=8000):
    return pl.pallas_call(
        _packtab_body,
        grid=(V // rt,),
        in_specs=[pl.BlockSpec((rt, D), lambda i: (i, 0))],
        out_specs=pl.BlockSpec((rt // 4, 128), lambda i: (i, 0)),
        out_shape=jax.ShapeDtypeStruct((V * (D // 2) // 128, 128), jnp.uint32),
    )(emb_table)


def _head_body(pooled_ref, mw_ref, clfw_ref, clfb_ref, out_ref):
    mw = mw_ref[...]
    fw = clfw_ref[...]
    logits = clfb_ref[...]
    for wdx in range(3):
        f = jnp.dot(
            fw[:, wdx * D : (wdx + 1) * D], mw, preferred_element_type=jnp.float32
        )
        logits = logits + jnp.dot(
            pooled_ref[wdx], f.T, preferred_element_type=jnp.float32
        )
    m = jnp.max(logits, axis=1, keepdims=True)
    e = jnp.exp(logits - m)
    out_ref[...] = e / jnp.sum(e, axis=1, keepdims=True)


def _head(pooled, m_w, clf_w, clf_b, bm=4096):
    return pl.pallas_call(
        _head_body,
        grid=(B // bm,),
        in_specs=[
            pl.BlockSpec((3, bm, D), lambda i: (0, i, 0)),
            pl.BlockSpec((D, D), lambda i: (0, 0)),
            pl.BlockSpec((O, 3 * D), lambda i: (0, 0)),
            pl.BlockSpec((1, O), lambda i: (0, 0)),
        ],
        out_specs=pl.BlockSpec((bm, O), lambda i: (i, 0)),
        out_shape=jax.ShapeDtypeStruct((B, O), jnp.float32),
    )(pooled, m_w, clf_w, clf_b)


def kernel(left_idx, term_idx, right_idx, emb_table, m_w, clf_w, clf_b):
    idx_all = jnp.concatenate(
        [
            left_idx.astype(jnp.int32),
            term_idx.astype(jnp.int32),
            right_idx.astype(jnp.int32),
        ],
        axis=0,
    )
    packed = _scpack(emb_table).reshape(V, D // 2)
    pooled = _pool(idx_all, packed).reshape(3, B, D)
    return _head(pooled, m_w, clf_w, clf_b.reshape(1, O))
